# baseline (speedup 1.0000x reference)
import functools

import jax
import jax.numpy as jnp
from jax import lax
from jax.experimental import pallas as pl
from jax.experimental.pallas import tpu as pltpu
from jax.experimental.pallas import tpu_sc as plsc

N = 100000
D = 128
L = 4096
P = 256
CPP = 16
H = 128

NW = 32
SEG_W = L // NW
CH = 512
NSUB = D // 16

N_CENTERS = L + P + 1
N_WORDS = 3 * N_CENTERS
GW = 512
N_WORDS_PAD = GW * NW


def _sc_body(feats_hbm, ids_hbm, starts_hbm, cflat_hbm, cidx_hbm,
             leaf_hbm, cout_hbm,
             starts_v, ids_v, rows_v, acc_v, cidx_v, cval_v, sem):
    wid = lax.axis_index("s") * 2 + lax.axis_index("c")
    base_seg = wid * SEG_W

    pltpu.sync_copy(cidx_hbm.at[pl.ds(wid * GW, GW)], cidx_v)
    for b in range(GW // 128):
        pltpu.async_copy(cflat_hbm.at[cidx_v.at[pl.ds(b * 128, 128)]],
                         cval_v.at[pl.ds(b * 128, 128)], sem).wait()
    pltpu.sync_copy(cval_v, cout_hbm.at[pl.ds(wid * GW, GW)])

    pltpu.sync_copy(starts_hbm.at[pl.ds(base_seg, 144)], starts_v)
    row_lo = starts_v[pl.ds(0, 16)][0]
    row_hi = starts_v[pl.ds(SEG_W, 16)][0]

    neg = jnp.full((16,), -jnp.inf, dtype=jnp.float32)

    def init_row(i, c):
        for j in range(NSUB):
            acc_v[i, pl.ds(16 * j, 16)] = neg
        return c
    lax.fori_loop(0, SEG_W + 1, init_row, 0)

    start0 = (row_lo // 8) * 8
    total = row_hi - start0
    nch = (total + CH - 1) // CH

    def chunk_body(k, c):
        start = jnp.minimum(start0 + k * CH, N - CH)
        pltpu.sync_copy(ids_hbm.at[pl.ds(start, CH)], ids_v)
        pltpu.sync_copy(feats_hbm.at[pl.ds(start, CH)], rows_v)

        def row_body(r16, cc):
            sv = ids_v[pl.ds(r16 * 16, 16)]
            for i in range(16):
                off = sv[i] - base_seg
                valid = (off >= 0) & (off < SEG_W)
                off = jnp.where(valid, off, SEG_W)
                r = r16 * 16 + i
                for j in range(NSUB):
                    v = rows_v[r, pl.ds(16 * j, 16)]
                    cur = acc_v[off, pl.ds(16 * j, 16)]
                    acc_v[off, pl.ds(16 * j, 16)] = jnp.maximum(cur, v)
            return cc
        lax.fori_loop(0, CH // 16, row_body, 0)
        return c
    lax.fori_loop(0, nch, chunk_body, 0)

    pltpu.sync_copy(acc_v.at[pl.ds(0, SEG_W)], leaf_hbm.at[pl.ds(base_seg, SEG_W)])


_sc_call = functools.partial(
    pl.kernel,
    out_type=[
        jax.ShapeDtypeStruct((L, D), jnp.float32),
        jax.ShapeDtypeStruct((N_WORDS_PAD,), jnp.float32),
    ],
    mesh=plsc.VectorSubcoreMesh(core_axis_name="c", subcore_axis_name="s"),
    scratch_types=[
        pltpu.VMEM((144,), jnp.int32),
        pltpu.VMEM((CH,), jnp.int32),
        pltpu.VMEM((CH, D), jnp.float32),
        pltpu.VMEM((SEG_W + 1, D), jnp.float32),
        pltpu.VMEM((GW,), jnp.int32),
        pltpu.VMEM((GW,), jnp.float32),
        pltpu.SemaphoreType.DMA,
    ],
)(_sc_body)


def _tc_mlp(leaf_ref, cp_ref, pp_ref, rp_ref, w1a_ref, w1b_ref, b1_ref,
            w2_ref, b2_ref, out1_ref, out0_ref):
    lf = leaf_ref[...]
    cp = cp_ref[...]
    pp = pp_ref[...]
    w1a = w1a_ref[...]
    w1b = w1b_ref[...]
    b1 = b1_ref[...]
    w2 = w2_ref[...]
    b2 = b2_ref[...]

    a = jnp.dot(lf, w1a, preferred_element_type=jnp.float32)
    ac = jnp.dot(cp, w1b, preferred_element_type=jnp.float32)
    bp = jnp.dot(pp, w1b, preferred_element_type=jnp.float32)
    bp_rep = jnp.broadcast_to(bp[:, None, :], (P, CPP, D)).reshape(L, D)
    h = jnp.maximum(a + ac - bp_rep + b1, 0.0)
    g = jnp.dot(h, w2, preferred_element_type=jnp.float32) + b2
    lvl1 = jnp.max(g.reshape(P, CPP, D), axis=1)
    out1_ref[...] = lvl1

    rel0 = pp - rp_ref[0:1, :]
    h0 = jnp.maximum(
        jnp.dot(lvl1, w1a, preferred_element_type=jnp.float32)
        + jnp.dot(rel0, w1b, preferred_element_type=jnp.float32) + b1, 0.0)
    g0 = jnp.dot(h0, w2, preferred_element_type=jnp.float32) + b2
    out0_ref[...] = jnp.max(g0, axis=0, keepdims=True)


@jax.jit
def kernel(precomputed_feats, coords, feats, leaf_ids, leaf_center_idx,
           parent_center_idx, root_center_idx, W1, b1, W2, b2):
    ids = leaf_ids.astype(jnp.int32)

    starts = jnp.searchsorted(
        ids, jnp.arange(L + 1, dtype=jnp.int32), side='left').astype(jnp.int32)
    starts_p = jnp.concatenate(
        [starts, jnp.full((15,), N, dtype=jnp.int32)])

    all_idx = jnp.concatenate([
        leaf_center_idx.astype(jnp.int32),
        parent_center_idx.astype(jnp.int32),
        root_center_idx.astype(jnp.int32),
    ])
    widx = (all_idx[:, None] * 3
            + jnp.arange(3, dtype=jnp.int32)[None, :]).reshape(-1)
    widx = jnp.concatenate(
        [widx, jnp.zeros((N_WORDS_PAD - N_WORDS,), dtype=jnp.int32)])

    leaf_feats, cwords = _sc_call(
        precomputed_feats, ids, starts_p, coords.reshape(-1), widx)

    cw = cwords[:N_WORDS].reshape(N_CENTERS, 3)
    cp4 = jnp.pad(cw[:L], ((0, 0), (0, 1)))
    pp4 = jnp.pad(cw[L:L + P], ((0, 0), (0, 1)))
    rp4 = jnp.pad(cw[L + P:], ((0, 7), (0, 1)))

    w1a = W1[:D]
    w1b = jnp.pad(W1[D:], ((0, 1), (0, 0)))

    level_1, level_0 = pl.pallas_call(
        _tc_mlp,
        out_shape=[
            jax.ShapeDtypeStruct((P, D), jnp.float32),
            jax.ShapeDtypeStruct((1, D), jnp.float32),
        ],
    )(leaf_feats, cp4, pp4, rp4, w1a, w1b, b1.reshape(1, D),
      W2, b2.reshape(1, D))

    return (level_0, level_1, leaf_feats)


# speedup vs baseline: 4.6176x; 4.6176x over previous
import functools

import jax
import jax.numpy as jnp
from jax import lax
from jax.experimental import pallas as pl
from jax.experimental.pallas import tpu as pltpu
from jax.experimental.pallas import tpu_sc as plsc

N = 100000
D = 128
L = 4096
P = 256
CPP = 16
H = 128

NW = 32
SEG_W = L // NW
CH = 512
NSUB = D // 16

G = 256
SAMPLE_N = (N + G - 1) // G
SAMPLE_PAD = 400


def _popcount(mask):
    r = plsc.all_reduce_population_count(mask)
    return r[0] if getattr(r, 'ndim', 0) else r


def _sc_body(feats_hbm, ids_hbm, sample_hbm, leaf_hbm,
             sample_v, ids_v, rows_v, acc_v):
    wid = lax.axis_index("s") * 2 + lax.axis_index("c")
    base_seg = wid * SEG_W

    pltpu.sync_copy(sample_hbm, sample_v)

    p_lo = jnp.int32(0)
    p_hi = jnp.int32(0)
    for t in range(SAMPLE_PAD // 16):
        sv = sample_v[pl.ds(16 * t, 16)]
        p_lo = p_lo + _popcount(sv < base_seg)
        p_hi = p_hi + _popcount(sv < base_seg + SEG_W)
    lo = jnp.maximum(p_lo - 1, 0) * G
    hi = jnp.minimum(p_hi * G, N)

    neg = jnp.full((16,), -jnp.inf, dtype=jnp.float32)

    def init_row(i, c):
        for j in range(NSUB):
            acc_v[i, pl.ds(16 * j, 16)] = neg
        return c
    lax.fori_loop(0, SEG_W + 1, init_row, 0)

    def merge_run(prev_sid, acc):
        off = prev_sid - base_seg
        valid = (off >= 0) & (off < SEG_W)
        off = jnp.where(valid, off, SEG_W)
        for j in range(NSUB):
            cur = acc_v[off, pl.ds(16 * j, 16)]
            acc_v[off, pl.ds(16 * j, 16)] = jnp.maximum(cur, acc[j])

    nch = (hi - lo + CH - 1) // CH

    def chunk_body(k, carry):
        start = jnp.minimum(lo + k * CH, N - CH)
        pltpu.sync_copy(ids_hbm.at[pl.ds(start, CH)], ids_v)
        pltpu.sync_copy(feats_hbm.at[pl.ds(start, CH)], rows_v)

        def row_body(r16, cc):
            prev_sid = cc[0]
            acc = list(cc[1:])
            sv = ids_v[pl.ds(r16 * 16, 16)]
            for i in range(16):
                sid = sv[i]
                r = r16 * 16 + i
                row = [rows_v[r, pl.ds(16 * j, 16)] for j in range(NSUB)]

                def flush(_):
                    merge_run(prev_sid, acc)
                    return tuple(row)

                def accum(_):
                    return tuple(jnp.maximum(acc[j], row[j])
                                 for j in range(NSUB))

                acc = list(lax.cond(sid != prev_sid, flush, accum, 0))
                prev_sid = sid
            return (prev_sid,) + tuple(acc)
        return lax.fori_loop(0, CH // 16, row_body, carry)

    init_carry = (jnp.int32(-2 * L),) + tuple(neg for _ in range(NSUB))
    final = lax.fori_loop(0, nch, chunk_body, init_carry)
    merge_run(final[0], list(final[1:]))

    pltpu.sync_copy(acc_v.at[pl.ds(0, SEG_W)],
                    leaf_hbm.at[pl.ds(base_seg, SEG_W)])


_sc_call = functools.partial(
    pl.kernel,
    out_type=jax.ShapeDtypeStruct((L, D), jnp.float32),
    mesh=plsc.VectorSubcoreMesh(core_axis_name="c", subcore_axis_name="s"),
    compiler_params=pltpu.CompilerParams(needs_layout_passes=False),
    scratch_types=[
        pltpu.VMEM((SAMPLE_PAD,), jnp.int32),
        pltpu.VMEM((CH,), jnp.int32),
        pltpu.VMEM((CH, D), jnp.float32),
        pltpu.VMEM((SEG_W + 1, D), jnp.float32),
    ],
)(_sc_body)


def _tc_mlp(leaf_ref, cp_ref, pp_ref, rp_ref, w1a_ref, w1b_ref, b1_ref,
            w2_ref, b2_ref, out1_ref, out0_ref):
    lf = leaf_ref[...]
    cp = cp_ref[...]
    pp = pp_ref[...]
    w1a = w1a_ref[...]
    w1b = w1b_ref[...]
    b1 = b1_ref[...]
    w2 = w2_ref[...]
    b2 = b2_ref[...]

    a = jnp.dot(lf, w1a, preferred_element_type=jnp.float32)
    ac = jnp.dot(cp, w1b, preferred_element_type=jnp.float32)
    bp = jnp.dot(pp, w1b, preferred_element_type=jnp.float32)
    bp_rep = jnp.broadcast_to(bp[:, None, :], (P, CPP, D)).reshape(L, D)
    h = jnp.maximum(a + ac - bp_rep + b1, 0.0)
    g = jnp.dot(h, w2, preferred_element_type=jnp.float32) + b2
    lvl1 = jnp.max(g.reshape(P, CPP, D), axis=1)
    out1_ref[...] = lvl1

    rel0 = pp - rp_ref[0:1, :]
    h0 = jnp.maximum(
        jnp.dot(lvl1, w1a, preferred_element_type=jnp.float32)
        + jnp.dot(rel0, w1b, preferred_element_type=jnp.float32) + b1, 0.0)
    g0 = jnp.dot(h0, w2, preferred_element_type=jnp.float32) + b2
    out0_ref[...] = jnp.max(g0, axis=0, keepdims=True)


@jax.jit
def kernel(precomputed_feats, coords, feats, leaf_ids, leaf_center_idx,
           parent_center_idx, root_center_idx, W1, b1, W2, b2):
    ids = leaf_ids.astype(jnp.int32)

    sample = jnp.concatenate([
        ids[::G],
        jnp.full((SAMPLE_PAD - SAMPLE_N,), jnp.int32(2 ** 30)),
    ])

    leaf_feats = _sc_call(precomputed_feats, ids, sample)

    cp4 = jnp.pad(coords[leaf_center_idx], ((0, 0), (0, 1)))
    pp4 = jnp.pad(coords[parent_center_idx], ((0, 0), (0, 1)))
    rp4 = jnp.pad(coords[root_center_idx], ((0, 7), (0, 1)))

    w1a = W1[:D]
    w1b = jnp.pad(W1[D:], ((0, 1), (0, 0)))

    level_1, level_0 = pl.pallas_call(
        _tc_mlp,
        out_shape=[
            jax.ShapeDtypeStruct((P, D), jnp.float32),
            jax.ShapeDtypeStruct((1, D), jnp.float32),
        ],
    )(leaf_feats, cp4, pp4, rp4, w1a, w1b, b1.reshape(1, D),
      W2, b2.reshape(1, D))

    return (level_0, level_1, leaf_feats)


# speedup vs baseline: 5.3011x; 1.1480x over previous
import functools

import jax
import jax.numpy as jnp
from jax import lax
from jax.experimental import pallas as pl
from jax.experimental.pallas import tpu as pltpu
from jax.experimental.pallas import tpu_sc as plsc

N = 100000
D = 128
L = 4096
P = 256
CPP = 16
H = 128

NW = 32
SEG_W = L // NW
CH = 384
NSUB = D // 16

G = 256
SAMPLE_N = (N + G - 1) // G
SAMPLE_PAD = 400


def _popcount(mask):
    r = plsc.all_reduce_population_count(mask)
    return r[0] if getattr(r, 'ndim', 0) else r


def _sc_body(feats_hbm, ids_hbm, sample_hbm, leaf_hbm,
             sample_v, ids_a, ids_b, rows_a, rows_b, acc_v, sem_a, sem_b):
    wid = lax.axis_index("s") * 2 + lax.axis_index("c")
    base_seg = wid * SEG_W

    pltpu.sync_copy(sample_hbm, sample_v)

    p_lo = jnp.int32(0)
    p_hi = jnp.int32(0)
    for t in range(SAMPLE_PAD // 16):
        sv = sample_v[pl.ds(16 * t, 16)]
        p_lo = p_lo + _popcount(sv < base_seg)
        p_hi = p_hi + _popcount(sv < base_seg + SEG_W)
    lo = jnp.maximum(p_lo - 1, 0) * G
    hi = jnp.minimum(p_hi * G, N)

    neg = jnp.full((16,), -jnp.inf, dtype=jnp.float32)

    def init_row(i, c):
        for j in range(NSUB):
            acc_v[i, pl.ds(16 * j, 16)] = neg
        return c
    lax.fori_loop(0, SEG_W + 1, init_row, 0)

    nch = (hi - lo + CH - 1) // CH

    def chunk_start(k):
        k = jnp.minimum(k, jnp.maximum(nch - 1, 0))
        return jnp.minimum(lo + k * CH, N - CH)

    def start_copy(k, ids_v, rows_v, sem):
        s = chunk_start(k)
        pltpu.make_async_copy(ids_hbm.at[pl.ds(s, CH)], ids_v, sem).start()
        pltpu.make_async_copy(feats_hbm.at[pl.ds(s, CH)], rows_v, sem).start()

    def wait_copy(ids_v, rows_v, sem):
        pltpu.make_async_copy(ids_hbm.at[pl.ds(0, CH)], ids_v, sem).wait()
        pltpu.make_async_copy(feats_hbm.at[pl.ds(0, CH)], rows_v, sem).wait()

    def merge_run(prev_sid, acc):
        off = prev_sid - base_seg
        valid = (off >= 0) & (off < SEG_W)
        off = jnp.where(valid, off, SEG_W)
        for j in range(NSUB):
            cur = acc_v[off, pl.ds(16 * j, 16)]
            acc_v[off, pl.ds(16 * j, 16)] = jnp.maximum(cur, acc[j])

    def process(ids_v, rows_v, carry):
        def row_body(r16, cc):
            prev_sid = cc[0]
            acc = list(cc[1:])
            sv = ids_v[pl.ds(r16 * 16, 16)]
            for i in range(16):
                sid = sv[i]
                changed = sid != prev_sid

                @pl.when(changed)
                def _():
                    merge_run(prev_sid, acc)

                f = jnp.where(changed, -jnp.inf, 0.0).astype(jnp.float32)
                r = r16 * 16 + i
                acc = [jnp.maximum(acc[j] + f, rows_v[r, pl.ds(16 * j, 16)])
                       for j in range(NSUB)]
                prev_sid = sid
            return (prev_sid,) + tuple(acc)
        return lax.fori_loop(0, CH // 16, row_body, carry)

    start_copy(0, ids_a, rows_a, sem_a)
    init_carry = (jnp.int32(-2 * L),) + tuple(neg for _ in range(NSUB))

    def pair_body(kk, carry):
        wait_copy(ids_a, rows_a, sem_a)
        start_copy(2 * kk + 1, ids_b, rows_b, sem_b)
        carry = process(ids_a, rows_a, carry)
        wait_copy(ids_b, rows_b, sem_b)
        start_copy(2 * kk + 2, ids_a, rows_a, sem_a)
        carry = process(ids_b, rows_b, carry)
        return carry

    npair = (nch + 1) // 2
    final = lax.fori_loop(0, npair, pair_body, init_carry)
    wait_copy(ids_a, rows_a, sem_a)
    merge_run(final[0], list(final[1:]))

    pltpu.sync_copy(acc_v.at[pl.ds(0, SEG_W)],
                    leaf_hbm.at[pl.ds(base_seg, SEG_W)])


_sc_call = functools.partial(
    pl.kernel,
    out_type=jax.ShapeDtypeStruct((L, D), jnp.float32),
    mesh=plsc.VectorSubcoreMesh(core_axis_name="c", subcore_axis_name="s"),
    compiler_params=pltpu.CompilerParams(needs_layout_passes=False,
                                         use_tc_tiling_on_sc=True),
    scratch_types=[
        pltpu.VMEM((SAMPLE_PAD,), jnp.int32),
        pltpu.VMEM((CH,), jnp.int32),
        pltpu.VMEM((CH,), jnp.int32),
        pltpu.VMEM((CH, D), jnp.float32),
        pltpu.VMEM((CH, D), jnp.float32),
        pltpu.VMEM((SEG_W + 1, D), jnp.float32),
        pltpu.SemaphoreType.DMA,
        pltpu.SemaphoreType.DMA,
    ],
)(_sc_body)


def _tc_mlp(leaf_ref, cp_ref, pp_ref, rp_ref, w1a_ref, w1b_ref, b1_ref,
            w2_ref, b2_ref, out1_ref, out0_ref):
    lf = leaf_ref[...]
    cp = cp_ref[...]
    pp = pp_ref[...]
    w1a = w1a_ref[...]
    w1b = w1b_ref[...]
    b1 = b1_ref[...]
    w2 = w2_ref[...]
    b2 = b2_ref[...]

    a = jnp.dot(lf, w1a, preferred_element_type=jnp.float32)
    ac = jnp.dot(cp, w1b, preferred_element_type=jnp.float32)
    bp = jnp.dot(pp, w1b, preferred_element_type=jnp.float32)
    bp_rep = jnp.broadcast_to(bp[:, None, :], (P, CPP, D)).reshape(L, D)
    h = jnp.maximum(a + ac - bp_rep + b1, 0.0)
    g = jnp.dot(h, w2, preferred_element_type=jnp.float32) + b2
    lvl1 = jnp.max(g.reshape(P, CPP, D), axis=1)
    out1_ref[...] = lvl1

    rel0 = pp - rp_ref[...]
    h0 = jnp.maximum(
        jnp.dot(lvl1, w1a, preferred_element_type=jnp.float32)
        + jnp.dot(rel0, w1b, preferred_element_type=jnp.float32) + b1, 0.0)
    g0 = jnp.dot(h0, w2, preferred_element_type=jnp.float32) + b2
    out0_ref[...] = jnp.max(g0, axis=0, keepdims=True)


@jax.jit
def kernel(precomputed_feats, coords, feats, leaf_ids, leaf_center_idx,
           parent_center_idx, root_center_idx, W1, b1, W2, b2):
    ids = leaf_ids.astype(jnp.int32)

    sample = jnp.concatenate([
        ids[::G],
        jnp.full((SAMPLE_PAD - SAMPLE_N,), jnp.int32(2 ** 30)),
    ])

    leaf_feats = _sc_call(precomputed_feats, ids, sample)

    cp3 = coords[leaf_center_idx]
    pp3 = coords[parent_center_idx]
    rp3 = coords[root_center_idx]

    w1a = W1[:D]
    w1b = W1[D:]

    level_1, level_0 = pl.pallas_call(
        _tc_mlp,
        out_shape=[
            jax.ShapeDtypeStruct((P, D), jnp.float32),
            jax.ShapeDtypeStruct((1, D), jnp.float32),
        ],
    )(leaf_feats, cp3, pp3, rp3, w1a, w1b, b1.reshape(1, D),
      W2, b2.reshape(1, D))

    return (level_0, level_1, leaf_feats)
